# baseline (speedup 1.0000x reference)
import functools

import jax
import jax.numpy as jnp
from jax import lax
from jax.experimental import pallas as pl
from jax.experimental.pallas import tpu as pltpu
import jax.experimental.pallas.tpu_sc as plsc

NC = 2
NS = 16
L = 16
NW = NC * NS

RD = 32
NT = 7


@functools.partial(jax.jit, static_argnums=(0,))
def _combine(B, tabs, idxs, w_bcast):
    b_per_w = B // NW
    mesh = plsc.VectorSubcoreMesh(core_axis_name="c", subcore_axis_name="s")

    @functools.partial(
        pl.kernel,
        out_type=jax.ShapeDtypeStruct((B, RD), jnp.float32),
        mesh=mesh,
        scratch_types=(
            [pltpu.VMEM((b_per_w,), jnp.int32) for _ in range(NT)]
            + [pltpu.VMEM((b_per_w, RD), jnp.float32) for _ in range(NT)]
            + [pltpu.VMEM((8, L), jnp.float32),
               pltpu.SemaphoreType.DMA, pltpu.SemaphoreType.DMA]
        ),
        compiler_params=pltpu.CompilerParams(use_tc_tiling_on_sc=False),
    )
    def k(t0, t1, t2, t3, t4, t5, t6, i0, i1, i2, i3, i4, i5, i6, w_hbm,
          out_hbm, x0, x1, x2, x3, x4, x5, x6, r0, r1, r2, r3, r4, r5, r6,
          w_v, sem, sem2):
        tab_refs = (t0, t1, t2, t3, t4, t5, t6)
        idx_refs = (i0, i1, i2, i3, i4, i5, i6)
        xv = (x0, x1, x2, x3, x4, x5, x6)
        rv = (r0, r1, r2, r3, r4, r5, r6)
        wid = lax.axis_index("s") * NC + lax.axis_index("c")
        base = wid * b_per_w

        idx_descs = [
            pltpu.async_copy(idx_refs[t].at[pl.ds(base, b_per_w)], xv[t], sem2)
            for t in range(NT)
        ]
        pltpu.sync_copy(w_hbm, w_v)
        for d in idx_descs:
            d.wait()
        CH = 128
        descs = [
            pltpu.async_copy(
                tab_refs[t].at[xv[t].at[pl.ds(j * CH, CH)]],
                rv[t].at[pl.ds(j * CH, CH)],
                sem,
            )
            for t in range(NT)
            for j in range(b_per_w // CH)
        ]
        for d in descs:
            d.wait()

        def body(g, carry):
            r0_ = g * 8
            for dr in range(8):
                r = r0_ + dr
                for c in (0, L):
                    acc = w_v[0] * rv[0][r, pl.ds(c, L)]
                    for t in range(1, NT):
                        acc = acc + w_v[t] * rv[t][r, pl.ds(c, L)]
                    rv[0][r, pl.ds(c, L)] = acc
            return carry

        lax.fori_loop(0, b_per_w // 8, body, 0)
        pltpu.sync_copy(rv[0], out_hbm.at[pl.ds(base, b_per_w)])

    return k(*tabs, *idxs, w_bcast)


def _mlp_body(x_ref, w1_ref, b1_ref, w2_ref, b2_ref, g_ref, be_ref, o_ref):
    x = x_ref[...]
    h = jnp.dot(x, w1_ref[...], preferred_element_type=jnp.float32,
                precision=lax.Precision.HIGHEST)
    h = jnp.maximum(h + b1_ref[...], 0.0)
    o = jnp.dot(h, w2_ref[...], preferred_element_type=jnp.float32,
                precision=lax.Precision.HIGHEST)
    o = o + b2_ref[...]
    mu = jnp.mean(o, axis=-1, keepdims=True)
    var = jnp.mean((o - mu) ** 2, axis=-1, keepdims=True)
    o_ref[...] = (o - mu) * lax.rsqrt(var + 1e-5) * g_ref[...] + be_ref[...]


@functools.partial(jax.jit, static_argnums=(0, 1))
def _mlp(B, blk, x, W1, b1, W2, b2, gamma, beta):
    H = W1.shape[1]
    ED = W2.shape[1]
    return pl.pallas_call(
        _mlp_body,
        grid=(B // blk,),
        in_specs=[
            pl.BlockSpec((blk, RD), lambda i: (i, 0)),
            pl.BlockSpec((RD, H), lambda i: (0, 0)),
            pl.BlockSpec((1, H), lambda i: (0, 0)),
            pl.BlockSpec((H, ED), lambda i: (0, 0)),
            pl.BlockSpec((1, ED), lambda i: (0, 0)),
            pl.BlockSpec((1, ED), lambda i: (0, 0)),
            pl.BlockSpec((1, ED), lambda i: (0, 0)),
        ],
        out_specs=pl.BlockSpec((blk, ED), lambda i: (i, 0)),
        out_shape=jax.ShapeDtypeStruct((B, ED), jnp.float32),
    )(x, W1, b1, W2, b2, gamma, beta)


def kernel(idx_phylum, idx_class, idx_order, idx_family, idx_genus,
           idx_species, tab_phylum, tab_class, tab_order, tab_family,
           tab_genus, tab_species, trophic_idx, trophic_tab, rank_weights,
           W1, b1, W2, b2, gamma, beta):
    B = idx_phylum.shape[0]
    idxs = [idx_phylum, idx_class, idx_order, idx_family, idx_genus,
            idx_species, trophic_idx]
    idxs = [i.astype(jnp.int32) for i in idxs]
    tabs = [tab_phylum, tab_class, tab_order, tab_family, tab_genus,
            tab_species, trophic_tab]
    w = jax.nn.softmax(rank_weights)
    wpad = jnp.zeros((8,), jnp.float32).at[:6].set(w).at[6].set(1.0)
    w_bcast = jnp.broadcast_to(wpad[:, None], (8, L))

    combined = _combine(B, tuple(tabs), tuple(idxs), w_bcast)
    out = _mlp(B, 2048, combined, W1, b1.reshape(1, -1), W2,
               b2.reshape(1, -1), gamma.reshape(1, -1), beta.reshape(1, -1))
    return out


# speedup vs baseline: 1.1066x; 1.1066x over previous
import functools

import jax
import jax.numpy as jnp
from jax import lax
from jax.experimental import pallas as pl
from jax.experimental.pallas import tpu as pltpu
import jax.experimental.pallas.tpu_sc as plsc

NC = 2
NS = 16
L = 16
NW = NC * NS

RD = 32

BIG = 4
SMALL = 3


@functools.partial(jax.jit, static_argnums=(0, 1))
def _combine(B, vocab_sizes, tabs, idxs, w_bcast):
    b_per_w = B // NW
    n_ord, n_fam, n_gen, n_spe, n_phy, n_cls, n_tro = vocab_sizes
    mesh = plsc.VectorSubcoreMesh(core_axis_name="c", subcore_axis_name="s")

    CH = 128

    @functools.partial(
        pl.kernel,
        out_type=jax.ShapeDtypeStruct((B, RD), jnp.float32),
        mesh=mesh,
        scratch_types=(
            [pltpu.VMEM((b_per_w,), jnp.int32) for _ in range(7)]
            + [pltpu.VMEM((b_per_w, RD), jnp.float32) for _ in range(BIG)]
            + [pltpu.VMEM((n_phy, RD), jnp.float32),
               pltpu.VMEM((n_cls, RD), jnp.float32),
               pltpu.VMEM((n_tro, RD), jnp.float32)]
            + [pltpu.VMEM_SHARED((n_ord, RD), jnp.float32),
               pltpu.VMEM_SHARED((n_fam, RD), jnp.float32)]
            + [pltpu.VMEM((8, L), jnp.float32),
               pltpu.SemaphoreType.DMA, pltpu.SemaphoreType.DMA]
        ),
        compiler_params=pltpu.CompilerParams(use_tc_tiling_on_sc=False,
                                             needs_layout_passes=False),
    )
    def k(t_ord, t_fam, t_gen, t_spe, t_phy, t_cls, t_tro,
          i_ord, i_fam, i_gen, i_spe, i_phy, i_cls, i_tro, w_hbm,
          out_hbm,
          x_ord, x_fam, x_gen, x_spe, x_phy, x_cls, x_tro,
          r_ord, r_fam, r_gen, r_spe,
          s_phy, s_cls, s_tro,
          sh_ord, sh_fam,
          w_v, sem, sem2):
        sid = lax.axis_index("s")
        wid = sid * NC + lax.axis_index("c")
        base = wid * b_per_w

        idx_refs = (i_ord, i_fam, i_gen, i_spe, i_phy, i_cls, i_tro)
        xv = (x_ord, x_fam, x_gen, x_spe, x_phy, x_cls, x_tro)
        rv = (r_ord, r_fam, r_gen, r_spe)

        idx_descs = [
            pltpu.async_copy(idx_refs[t].at[pl.ds(base, b_per_w)], xv[t], sem2)
            for t in range(7)
        ]
        pltpu.sync_copy(w_hbm, w_v)
        for d in idx_descs:
            d.wait()

        spe_descs = [
            pltpu.async_copy(
                tab.at[x.at[pl.ds(j * CH, CH)]],
                r.at[pl.ds(j * CH, CH)],
                sem,
            )
            for tab, x, r in ((t_spe, x_spe, r_spe), (t_gen, x_gen, r_gen))
            for j in range(b_per_w // CH)
        ]

        pltpu.sync_copy(t_phy, s_phy)
        pltpu.sync_copy(t_cls, s_cls)
        pltpu.sync_copy(t_tro, s_tro)

        for tab_hbm, sh_ref, n_rows in (
            (t_ord, sh_ord, n_ord), (t_fam, sh_fam, n_fam),
        ):
            chunk = (n_rows + NS - 1) // NS
            start = jnp.minimum(sid * chunk, n_rows - chunk)
            pltpu.sync_copy(tab_hbm.at[pl.ds(start, chunk)],
                            sh_ref.at[pl.ds(start, chunk)])
        plsc.subcore_barrier()

        mid_descs = [
            pltpu.async_copy(
                sh.at[x.at[pl.ds(j * CH, CH)]],
                r.at[pl.ds(j * CH, CH)],
                sem,
            )
            for sh, x, r in ((sh_ord, x_ord, r_ord), (sh_fam, x_fam, r_fam))
            for j in range(b_per_w // CH)
        ]
        for d in spe_descs:
            d.wait()
        for d in mid_descs:
            d.wait()

        def body(g, carry):
            r0_ = g * 8
            for dr in range(8):
                r = r0_ + dr
                for c in (0, L):
                    acc = w_v[0] * rv[0][r, pl.ds(c, L)]
                    for t in range(1, BIG):
                        acc = acc + w_v[t] * rv[t][r, pl.ds(c, L)]
                    rv[0][r, pl.ds(c, L)] = acc
            return carry

        lax.fori_loop(0, b_per_w // 8, body, 0)

        lanes = lax.iota(jnp.int32, L)

        def small_body(g, carry):
            rows = g * L + lanes
            for t, s_ref in ((4, s_phy), (5, s_cls), (6, s_tro)):
                idx16 = xv[t][pl.ds(g * L, L)]
                for d in range(RD):
                    col = jnp.full((L,), d, jnp.int32)
                    vals = plsc.load_gather(s_ref, [idx16, col])
                    plsc.addupdate_scatter(r_ord, [rows, col],
                                           w_v[t] * vals)
            return carry

        lax.fori_loop(0, b_per_w // L, small_body, 0)

        pltpu.sync_copy(r_ord, out_hbm.at[pl.ds(base, b_per_w)])

    return k(*tabs, *idxs, w_bcast)


def _mlp_body(x_ref, w1_ref, b1_ref, w2_ref, b2_ref, g_ref, be_ref, o_ref):
    x = x_ref[...]
    h = jnp.dot(x, w1_ref[...], preferred_element_type=jnp.float32,
                precision=lax.Precision.HIGHEST)
    h = jnp.maximum(h + b1_ref[...], 0.0)
    o = jnp.dot(h, w2_ref[...], preferred_element_type=jnp.float32,
                precision=lax.Precision.HIGHEST)
    o = o + b2_ref[...]
    mu = jnp.mean(o, axis=-1, keepdims=True)
    var = jnp.mean((o - mu) ** 2, axis=-1, keepdims=True)
    o_ref[...] = (o - mu) * lax.rsqrt(var + 1e-5) * g_ref[...] + be_ref[...]


@functools.partial(jax.jit, static_argnums=(0, 1))
def _mlp(B, blk, x, W1, b1, W2, b2, gamma, beta):
    H = W1.shape[1]
    ED = W2.shape[1]
    return pl.pallas_call(
        _mlp_body,
        grid=(B // blk,),
        in_specs=[
            pl.BlockSpec((blk, RD), lambda i: (i, 0)),
            pl.BlockSpec((RD, H), lambda i: (0, 0)),
            pl.BlockSpec((1, H), lambda i: (0, 0)),
            pl.BlockSpec((H, ED), lambda i: (0, 0)),
            pl.BlockSpec((1, ED), lambda i: (0, 0)),
            pl.BlockSpec((1, ED), lambda i: (0, 0)),
            pl.BlockSpec((1, ED), lambda i: (0, 0)),
        ],
        out_specs=pl.BlockSpec((blk, ED), lambda i: (i, 0)),
        out_shape=jax.ShapeDtypeStruct((B, ED), jnp.float32),
    )(x, W1, b1, W2, b2, gamma, beta)


def kernel(idx_phylum, idx_class, idx_order, idx_family, idx_genus,
           idx_species, tab_phylum, tab_class, tab_order, tab_family,
           tab_genus, tab_species, trophic_idx, trophic_tab, rank_weights,
           W1, b1, W2, b2, gamma, beta):
    B = idx_phylum.shape[0]
    idxs = [idx_order, idx_family, idx_genus, idx_species,
            idx_phylum, idx_class, trophic_idx]
    idxs = [i.astype(jnp.int32) for i in idxs]
    tabs = [tab_order, tab_family, tab_genus, tab_species,
            tab_phylum, tab_class, trophic_tab]
    w = jax.nn.softmax(rank_weights)
    wre = jnp.stack([w[2], w[3], w[4], w[5], w[0], w[1],
                     jnp.float32(1.0), jnp.float32(0.0)])
    w_bcast = jnp.broadcast_to(wre[:, None], (8, L))
    vocab_sizes = tuple(int(t.shape[0]) for t in tabs)

    combined = _combine(B, vocab_sizes, tuple(tabs), tuple(idxs), w_bcast)
    out = _mlp(B, 2048, combined, W1, b1.reshape(1, -1), W2,
               b2.reshape(1, -1), gamma.reshape(1, -1), beta.reshape(1, -1))
    return out
